# baseline (speedup 1.0000x reference)
import functools

import jax
import jax.numpy as jnp
from jax import lax
from jax.experimental import pallas as pl
from jax.experimental.pallas import tpu as pltpu
from jax.experimental.pallas import tpu_sc as plsc

N = 10000
NP = 10240
E = 320000
D = 128
DE = 16

NC = 2
NS = 16
NW = NC * NS
EPW = E // NW
CH = 40
NCHUNK = EPW // CH
ROWS_PER_SUB = NP // NS
ZROWS = 32


def _p_body(x_ref, w_ref, o_ref):
    o_ref[...] = jnp.dot(x_ref[...], w_ref[...],
                         preferred_element_type=jnp.float32)


def _q_body(e_ref, w_ref, b_ref, o_ref):
    o_ref[...] = jnp.dot(e_ref[...], w_ref[...],
                         preferred_element_type=jnp.float32) + b_ref[...]


def _edge_out_body(e_ref, dep_ref, o_ref):
    o_ref[...] = e_ref[...]


def _upd_body(x_ref, agg_ref, w1_ref, w2_ref, b_ref, o_ref):
    agg = agg_ref[0] + agg_ref[1]
    o_ref[...] = (
        jnp.dot(x_ref[...], w1_ref[...], preferred_element_type=jnp.float32)
        + jnp.dot(agg, w2_ref[...], preferred_element_type=jnp.float32)
        + b_ref[...]
    )


def _compute_p(node_latents, w1):
    blk = 2000
    return pl.pallas_call(
        _p_body,
        grid=(N // blk,),
        in_specs=[
            pl.BlockSpec((blk, D), lambda i: (i, 0)),
            pl.BlockSpec((D, D), lambda i: (0, 0)),
        ],
        out_specs=pl.BlockSpec((blk, D), lambda i: (i, 0)),
        out_shape=jax.ShapeDtypeStruct((NP, D), jnp.float32),
    )(node_latents, w1)


def _compute_q(edge_latents, w2, b_msg):
    blk = 8000
    return pl.pallas_call(
        _q_body,
        grid=(E // blk,),
        in_specs=[
            pl.BlockSpec((blk, DE), lambda i: (i, 0)),
            pl.BlockSpec((DE, D), lambda i: (0, 0)),
            pl.BlockSpec((1, D), lambda i: (0, 0)),
        ],
        out_specs=pl.BlockSpec((blk, D), lambda i: (i, 0)),
        out_shape=jax.ShapeDtypeStruct((E, D), jnp.float32),
    )(edge_latents.astype(jnp.bfloat16), w2.astype(jnp.bfloat16),
      b_msg.reshape(1, D))


def _edge_passthrough(edge_latents, agg_partials):
    blk = 16000
    return pl.pallas_call(
        _edge_out_body,
        grid=(E // blk,),
        in_specs=[
            pl.BlockSpec((blk, DE), lambda i: (i, 0)),
            pl.BlockSpec((1, 8, D), lambda i: (0, 0, 0)),
        ],
        out_specs=pl.BlockSpec((blk, DE), lambda i: (i, 0)),
        out_shape=jax.ShapeDtypeStruct((E, DE), jnp.float32),
    )(edge_latents, agg_partials)


def _compute_update(node_latents, agg_partials, wu1, wu2, b_upd):
    blk = 2000
    return pl.pallas_call(
        _upd_body,
        grid=(N // blk,),
        in_specs=[
            pl.BlockSpec((blk, D), lambda i: (i, 0)),
            pl.BlockSpec((NC, blk, D), lambda i: (0, i, 0)),
            pl.BlockSpec((D, D), lambda i: (0, 0)),
            pl.BlockSpec((D, D), lambda i: (0, 0)),
            pl.BlockSpec((1, D), lambda i: (0, 0)),
        ],
        out_specs=pl.BlockSpec((blk, D), lambda i: (i, 0)),
        out_shape=jax.ShapeDtypeStruct((N, D), jnp.float32),
    )(node_latents, agg_partials, wu1, wu2, b_upd.reshape(1, D))


NSLOT = 4


def _sc_body(p_hbm, q_hbm, send_hbm, recv_hbm, out_hbm, *refs):
    sidxs = refs[0:NSLOT]
    ridxs = refs[NSLOT:2 * NSLOT]
    prows = refs[2 * NSLOT:3 * NSLOT]
    qrows = refs[3 * NSLOT:4 * NSLOT]
    zbuf = refs[4 * NSLOT]
    agg_sh = refs[4 * NSLOT + 1]
    semi = refs[4 * NSLOT + 2:4 * NSLOT + 2 + NSLOT]
    semj = refs[4 * NSLOT + 2 + NSLOT:4 * NSLOT + 2 + 2 * NSLOT]
    semg = refs[4 * NSLOT + 2 + 2 * NSLOT:4 * NSLOT + 2 + 3 * NSLOT]
    semq = refs[4 * NSLOT + 2 + 3 * NSLOT:4 * NSLOT + 2 + 4 * NSLOT]

    cid = lax.axis_index("c")
    sid = lax.axis_index("s")
    wid = cid * NS + sid

    def zero_body(t, _):
        i = t // 8
        j = (t % 8) * 16
        zbuf[i, pl.ds(j, 16)] = jnp.zeros((16,), jnp.float32)
        return _
    lax.fori_loop(0, ZROWS * 8, zero_body, None)
    base_row = sid * ROWS_PER_SUB
    for r in range(ROWS_PER_SUB // ZROWS):
        pltpu.sync_copy(zbuf, agg_sh.at[pl.ds(base_row + r * ZROWS, ZROWS)])
    plsc.subcore_barrier()

    def issue_idx(c, s):
        off = wid * EPW + c * CH
        pltpu.async_copy(send_hbm.at[pl.ds(off, CH)], sidxs[s], semi[s])
        pltpu.async_copy(recv_hbm.at[pl.ds(off, CH)], ridxs[s], semj[s])

    def issue_data(c, s):
        off = wid * EPW + c * CH
        pltpu.make_async_copy(send_hbm.at[pl.ds(off, CH)], sidxs[s],
                              semi[s]).wait()
        pltpu.async_copy(p_hbm.at[sidxs[s]], prows[s], semg[s])
        pltpu.async_copy(q_hbm.at[pl.ds(off, CH)], qrows[s], semq[s])

    def step(c, s):
        off = wid * EPW + c * CH
        pltpu.make_async_copy(p_hbm.at[sidxs[s]], prows[s], semg[s]).wait()
        pltpu.make_async_copy(q_hbm.at[pl.ds(off, CH)], qrows[s],
                              semq[s]).wait()

        def comp(i, _):
            for j in range(8):
                sl = pl.ds(j * 16, 16)
                prows[s][i, sl] = jnp.maximum(prows[s][i, sl] + qrows[s][i, sl],
                                              0.0)
            return _
        lax.fori_loop(0, CH, comp, None)
        pltpu.make_async_copy(recv_hbm.at[pl.ds(off, CH)], ridxs[s],
                              semj[s]).wait()
        pltpu.sync_copy(prows[s], agg_sh.at[ridxs[s]], add=True)

    for c in range(NSLOT):
        issue_idx(c, c)
    issue_data(0, 0)
    issue_data(1, 1)

    nmain = (NCHUNK - NSLOT - 1) // NSLOT

    def loop_body(kk, _):
        b = kk * NSLOT
        for s in range(NSLOT):
            c = b + s
            step(c, s)
            issue_idx(c + NSLOT, s)
            issue_data(c + 2, (s + 2) % NSLOT)
        return _
    lax.fori_loop(0, nmain, loop_body, None)

    for c in range(nmain * NSLOT, NCHUNK):
        s = c % NSLOT
        step(c, s)
        if c + NSLOT < NCHUNK:
            issue_idx(c + NSLOT, s)
        if c + 2 < NCHUNK:
            issue_data(c + 2, (c + 2) % NSLOT)

    plsc.subcore_barrier()

    for r in range(ROWS_PER_SUB // ZROWS):
        row = base_row + r * ZROWS
        pltpu.sync_copy(agg_sh.at[pl.ds(row, ZROWS)], zbuf)
        pltpu.sync_copy(zbuf, out_hbm.at[cid, pl.ds(row, ZROWS)])


def _sc_aggregate(p, q, senders, receivers):
    mesh = plsc.VectorSubcoreMesh(core_axis_name="c", subcore_axis_name="s")
    scratch = (
        [pltpu.VMEM((CH,), jnp.int32) for _ in range(2 * NSLOT)]
        + [pltpu.VMEM((CH, D), jnp.float32) for _ in range(2 * NSLOT)]
        + [pltpu.VMEM((ZROWS, D), jnp.float32),
           pltpu.VMEM_SHARED((NP, D), jnp.float32)]
        + [pltpu.SemaphoreType.DMA for _ in range(4 * NSLOT)]
    )
    kern = functools.partial(
        pl.kernel,
        mesh=mesh,
        out_type=jax.ShapeDtypeStruct((NC, NP, D), jnp.float32),
        scratch_types=scratch,
    )(_sc_body)
    return kern(p, q, senders, receivers)


@jax.jit
def kernel(node_latents, edge_latents, edge_index, W_msg, b_msg, W_upd, b_upd):
    w1 = W_msg[:D]
    w2 = W_msg[D:]
    wu1 = W_upd[:D]
    wu2 = W_upd[D:]

    p = _compute_p(node_latents, w1)
    q = _compute_q(edge_latents, w2, b_msg)
    agg_partials = _sc_aggregate(p, q, edge_index[0], edge_index[1])
    new_node_latents = _compute_update(node_latents, agg_partials, wu1, wu2,
                                       b_upd)
    edge_out = _edge_passthrough(edge_latents, agg_partials)
    return (new_node_latents, edge_out)


# speedup vs baseline: 1.4517x; 1.4517x over previous
import functools

import jax
import jax.numpy as jnp
from jax import lax
from jax.experimental import pallas as pl
from jax.experimental.pallas import tpu as pltpu
from jax.experimental.pallas import tpu_sc as plsc

N = 10000
NP = 10240
E = 320000
D = 128
DE = 16

NC = 2
NS = 16
NW = NC * NS
EPW = E // NW
CH = 40
NCHUNK = EPW // CH
ROWS_PER_SUB = NP // NS
ZROWS = 32


def _p_body(x_ref, w_ref, o_ref):
    o_ref[...] = jnp.dot(x_ref[...], w_ref[...],
                         preferred_element_type=jnp.float32)


def _q_body(e_ref, w_ref, b_ref, o_ref):
    o_ref[...] = jnp.dot(e_ref[...], w_ref[...],
                         preferred_element_type=jnp.float32) + b_ref[...]


def _edge_out_body(e_ref, dep_ref, o_ref):
    o_ref[...] = e_ref[...]


def _upd_body(x_ref, agg_ref, w1_ref, w2_ref, b_ref, o_ref):
    agg = agg_ref[0] + agg_ref[1]
    o_ref[...] = (
        jnp.dot(x_ref[...], w1_ref[...], preferred_element_type=jnp.float32)
        + jnp.dot(agg, w2_ref[...], preferred_element_type=jnp.float32)
        + b_ref[...]
    )


def _compute_p(node_latents, w1):
    blk = 2000
    return pl.pallas_call(
        _p_body,
        grid=(N // blk,),
        in_specs=[
            pl.BlockSpec((blk, D), lambda i: (i, 0)),
            pl.BlockSpec((D, D), lambda i: (0, 0)),
        ],
        out_specs=pl.BlockSpec((blk, D), lambda i: (i, 0)),
        out_shape=jax.ShapeDtypeStruct((NP, D), jnp.float32),
    )(node_latents, w1)


def _compute_q(edge_latents, w2, b_msg):
    blk = 8000
    return pl.pallas_call(
        _q_body,
        grid=(E // blk,),
        in_specs=[
            pl.BlockSpec((blk, DE), lambda i: (i, 0)),
            pl.BlockSpec((DE, D), lambda i: (0, 0)),
            pl.BlockSpec((1, D), lambda i: (0, 0)),
        ],
        out_specs=pl.BlockSpec((blk, D), lambda i: (i, 0)),
        out_shape=jax.ShapeDtypeStruct((E, D), jnp.float32),
    )(edge_latents.astype(jnp.bfloat16), w2.astype(jnp.bfloat16),
      b_msg.reshape(1, D))


def _edge_passthrough(edge_latents, agg_partials):
    blk = 16000
    return pl.pallas_call(
        _edge_out_body,
        grid=(E // blk,),
        in_specs=[
            pl.BlockSpec((blk, DE), lambda i: (i, 0)),
            pl.BlockSpec((1, 8, D), lambda i: (0, 0, 0)),
        ],
        out_specs=pl.BlockSpec((blk, DE), lambda i: (i, 0)),
        out_shape=jax.ShapeDtypeStruct((E, DE), jnp.float32),
    )(edge_latents, agg_partials)


def _compute_update(node_latents, agg_partials, wu1, wu2, b_upd):
    blk = 2000
    return pl.pallas_call(
        _upd_body,
        grid=(N // blk,),
        in_specs=[
            pl.BlockSpec((blk, D), lambda i: (i, 0)),
            pl.BlockSpec((NC, blk, D), lambda i: (0, i, 0)),
            pl.BlockSpec((D, D), lambda i: (0, 0)),
            pl.BlockSpec((D, D), lambda i: (0, 0)),
            pl.BlockSpec((1, D), lambda i: (0, 0)),
        ],
        out_specs=pl.BlockSpec((blk, D), lambda i: (i, 0)),
        out_shape=jax.ShapeDtypeStruct((N, D), jnp.float32),
    )(node_latents, agg_partials, wu1, wu2, b_upd.reshape(1, D))


NSLOT = 4


def _sc_body(p_hbm, q_hbm, send_hbm, recv_hbm, out_hbm, *refs):
    sidxs = refs[0:NSLOT]
    ridxs = refs[NSLOT:2 * NSLOT]
    prows = refs[2 * NSLOT:3 * NSLOT]
    qrows = refs[3 * NSLOT:4 * NSLOT]
    zbuf = refs[4 * NSLOT]
    agg_sh = refs[4 * NSLOT + 1]
    semi = refs[4 * NSLOT + 2:4 * NSLOT + 2 + NSLOT]
    semj = refs[4 * NSLOT + 2 + NSLOT:4 * NSLOT + 2 + 2 * NSLOT]
    semg = refs[4 * NSLOT + 2 + 2 * NSLOT:4 * NSLOT + 2 + 3 * NSLOT]
    semq = refs[4 * NSLOT + 2 + 3 * NSLOT:4 * NSLOT + 2 + 4 * NSLOT]

    cid = lax.axis_index("c")
    sid = lax.axis_index("s")
    wid = cid * NS + sid

    def zero_body(t, _):
        i = t // 8
        j = (t % 8) * 16
        zbuf[i, pl.ds(j, 16)] = jnp.zeros((16,), jnp.float32)
        return _
    lax.fori_loop(0, ZROWS * 8, zero_body, None)
    base_row = sid * ROWS_PER_SUB
    for r in range(ROWS_PER_SUB // ZROWS):
        pltpu.sync_copy(zbuf, agg_sh.at[pl.ds(base_row + r * ZROWS, ZROWS)])
    plsc.subcore_barrier()

    def issue_idx(c, s):
        off = wid * EPW + c * CH
        pltpu.async_copy(send_hbm.at[pl.ds(off, CH)], sidxs[s], semi[s])
        pltpu.async_copy(recv_hbm.at[pl.ds(off, CH)], ridxs[s], semj[s])

    def issue_data(c, s):
        off = wid * EPW + c * CH
        pltpu.make_async_copy(send_hbm.at[pl.ds(off, CH)], sidxs[s],
                              semi[s]).wait()
        pltpu.async_copy(p_hbm.at[sidxs[s]], prows[s], semg[s])
        pltpu.async_copy(q_hbm.at[pl.ds(off, CH)], qrows[s], semq[s])

    def step(c, s):
        off = wid * EPW + c * CH
        pltpu.make_async_copy(p_hbm.at[sidxs[s]], prows[s], semg[s]).wait()
        pltpu.make_async_copy(q_hbm.at[pl.ds(off, CH)], qrows[s],
                              semq[s]).wait()

        def comp(i, _):
            for j in range(8):
                sl = pl.ds(j * 16, 16)
                prows[s][i, sl] = jnp.maximum(prows[s][i, sl] + qrows[s][i, sl],
                                              0.0)
            return _
        lax.fori_loop(0, CH, comp, None)
        pltpu.make_async_copy(recv_hbm.at[pl.ds(off, CH)], ridxs[s],
                              semj[s]).wait()
        pltpu.sync_copy(prows[s], agg_sh.at[ridxs[s]], add=True)

    for c in range(NSLOT):
        issue_idx(c, c)
    issue_data(0, 0)
    issue_data(1, 1)

    nmain = (NCHUNK - NSLOT - 1) // NSLOT

    def loop_body(kk, _):
        b = kk * NSLOT
        for s in range(NSLOT):
            c = b + s
            step(c, s)
            issue_idx(c + NSLOT, s)
            issue_data(c + 2, (s + 2) % NSLOT)
        return _
    lax.fori_loop(0, nmain, loop_body, None)

    for c in range(nmain * NSLOT, NCHUNK):
        s = c % NSLOT
        step(c, s)
        if c + NSLOT < NCHUNK:
            issue_idx(c + NSLOT, s)
        if c + 2 < NCHUNK:
            issue_data(c + 2, (c + 2) % NSLOT)

    plsc.subcore_barrier()

    for r in range(ROWS_PER_SUB // ZROWS):
        row = base_row + r * ZROWS
        pltpu.sync_copy(agg_sh.at[pl.ds(row, ZROWS)], zbuf)
        pltpu.sync_copy(zbuf, out_hbm.at[cid, pl.ds(row, ZROWS)])


def _sc_aggregate(p, q, senders, receivers):
    mesh = plsc.VectorSubcoreMesh(core_axis_name="c", subcore_axis_name="s")
    scratch = (
        [pltpu.VMEM((CH,), jnp.int32) for _ in range(2 * NSLOT)]
        + [pltpu.VMEM((CH, D), jnp.float32) for _ in range(2 * NSLOT)]
        + [pltpu.VMEM((ZROWS, D), jnp.float32),
           pltpu.VMEM_SHARED((NP, D), jnp.float32)]
        + [pltpu.SemaphoreType.DMA for _ in range(4 * NSLOT)]
    )
    kern = functools.partial(
        pl.kernel,
        mesh=mesh,
        out_type=jax.ShapeDtypeStruct((NC, NP, D), jnp.float32),
        scratch_types=scratch,
    )(_sc_body)
    return kern(p, q, senders, receivers)


@jax.jit
def kernel(node_latents, edge_latents, edge_index, W_msg, b_msg, W_upd, b_upd):
    w1 = W_msg[:D]
    w2 = W_msg[D:]
    wu1 = W_upd[:D]
    wu2 = W_upd[D:]

    p = _compute_p(node_latents, w1)
    q = _compute_q(edge_latents, w2, b_msg)
    agg_partials = _sc_aggregate(p, q, edge_index[0], edge_index[1])
    new_node_latents = _compute_update(node_latents, agg_partials, wu1, wu2,
                                       b_upd)
    return (new_node_latents, edge_latents)


# speedup vs baseline: 1.8050x; 1.2433x over previous
import functools

import jax
import jax.numpy as jnp
from jax import lax
from jax.experimental import pallas as pl
from jax.experimental.pallas import tpu as pltpu
from jax.experimental.pallas import tpu_sc as plsc

N = 10000
NP = 10240
E = 320000
D = 128
DE = 16

NC = 2
NS = 16
NW = NC * NS
EPW = E // NW
CH = 40
NCHUNK = EPW // CH
ROWS_PER_SUB = NP // NS
ZROWS = 32


def _p_body(x_ref, w_ref, o_ref):
    o_ref[...] = jnp.dot(x_ref[...], w_ref[...],
                         preferred_element_type=jnp.float32)


def _q_body(e_ref, w_ref, b_ref, o_ref):
    o_ref[...] = lax.dot_general(
        e_ref[...], w_ref[...],
        dimension_numbers=(((0,), (0,)), ((), ())),
        preferred_element_type=jnp.float32) + b_ref[...]


def _edge_out_body(e_ref, dep_ref, o_ref):
    o_ref[...] = e_ref[...]


def _upd_body(x_ref, agg_ref, w1_ref, w2_ref, b_ref, o_ref):
    agg = agg_ref[0] + agg_ref[1]
    o_ref[...] = (
        jnp.dot(x_ref[...], w1_ref[...], preferred_element_type=jnp.float32)
        + jnp.dot(agg, w2_ref[...], preferred_element_type=jnp.float32)
        + b_ref[...]
    )


def _compute_p(node_latents, w1):
    blk = 2000
    return pl.pallas_call(
        _p_body,
        grid=(N // blk,),
        in_specs=[
            pl.BlockSpec((blk, D), lambda i: (i, 0)),
            pl.BlockSpec((D, D), lambda i: (0, 0)),
        ],
        out_specs=pl.BlockSpec((blk, D), lambda i: (i, 0)),
        out_shape=jax.ShapeDtypeStruct((NP, D), jnp.float32),
    )(node_latents, w1)


def _compute_q(edge_latents, w2, b_msg):
    blk = 16000
    return pl.pallas_call(
        _q_body,
        grid=(E // blk,),
        in_specs=[
            pl.BlockSpec((DE, blk), lambda i: (0, i)),
            pl.BlockSpec((DE, D), lambda i: (0, 0)),
            pl.BlockSpec((1, D), lambda i: (0, 0)),
        ],
        out_specs=pl.BlockSpec((blk, D), lambda i: (i, 0)),
        out_shape=jax.ShapeDtypeStruct((E, D), jnp.float32),
    )(edge_latents.T.astype(jnp.bfloat16), w2.astype(jnp.bfloat16),
      b_msg.reshape(1, D))


def _edge_passthrough(edge_latents, agg_partials):
    blk = 16000
    return pl.pallas_call(
        _edge_out_body,
        grid=(E // blk,),
        in_specs=[
            pl.BlockSpec((blk, DE), lambda i: (i, 0)),
            pl.BlockSpec((1, 8, D), lambda i: (0, 0, 0)),
        ],
        out_specs=pl.BlockSpec((blk, DE), lambda i: (i, 0)),
        out_shape=jax.ShapeDtypeStruct((E, DE), jnp.float32),
    )(edge_latents, agg_partials)


def _compute_update(node_latents, agg_partials, wu1, wu2, b_upd):
    blk = 2000
    return pl.pallas_call(
        _upd_body,
        grid=(N // blk,),
        in_specs=[
            pl.BlockSpec((blk, D), lambda i: (i, 0)),
            pl.BlockSpec((NC, blk, D), lambda i: (0, i, 0)),
            pl.BlockSpec((D, D), lambda i: (0, 0)),
            pl.BlockSpec((D, D), lambda i: (0, 0)),
            pl.BlockSpec((1, D), lambda i: (0, 0)),
        ],
        out_specs=pl.BlockSpec((blk, D), lambda i: (i, 0)),
        out_shape=jax.ShapeDtypeStruct((N, D), jnp.float32),
    )(node_latents, agg_partials, wu1, wu2, b_upd.reshape(1, D))


NSLOT = 4


def _sc_body(p_hbm, q_hbm, send_hbm, recv_hbm, out_hbm, *refs):
    sidxs = refs[0:NSLOT]
    ridxs = refs[NSLOT:2 * NSLOT]
    prows = refs[2 * NSLOT:3 * NSLOT]
    qrows = refs[3 * NSLOT:4 * NSLOT]
    zbuf = refs[4 * NSLOT]
    agg_sh = refs[4 * NSLOT + 1]
    semi = refs[4 * NSLOT + 2:4 * NSLOT + 2 + NSLOT]
    semj = refs[4 * NSLOT + 2 + NSLOT:4 * NSLOT + 2 + 2 * NSLOT]
    semg = refs[4 * NSLOT + 2 + 2 * NSLOT:4 * NSLOT + 2 + 3 * NSLOT]
    semq = refs[4 * NSLOT + 2 + 3 * NSLOT:4 * NSLOT + 2 + 4 * NSLOT]

    cid = lax.axis_index("c")
    sid = lax.axis_index("s")
    wid = cid * NS + sid

    def zero_body(t, _):
        i = t // 8
        j = (t % 8) * 16
        zbuf[i, pl.ds(j, 16)] = jnp.zeros((16,), jnp.float32)
        return _
    lax.fori_loop(0, ZROWS * 8, zero_body, None)
    base_row = sid * ROWS_PER_SUB
    for r in range(ROWS_PER_SUB // ZROWS):
        pltpu.sync_copy(zbuf, agg_sh.at[pl.ds(base_row + r * ZROWS, ZROWS)])
    plsc.subcore_barrier()

    def issue_idx(c, s):
        off = wid * EPW + c * CH
        pltpu.async_copy(send_hbm.at[pl.ds(off, CH)], sidxs[s], semi[s])
        pltpu.async_copy(recv_hbm.at[pl.ds(off, CH)], ridxs[s], semj[s])

    def issue_data(c, s):
        off = wid * EPW + c * CH
        pltpu.make_async_copy(send_hbm.at[pl.ds(off, CH)], sidxs[s],
                              semi[s]).wait()
        pltpu.async_copy(p_hbm.at[sidxs[s]], prows[s], semg[s])
        pltpu.async_copy(q_hbm.at[pl.ds(off, CH)], qrows[s], semq[s])

    def step(c, s):
        off = wid * EPW + c * CH
        pltpu.make_async_copy(p_hbm.at[sidxs[s]], prows[s], semg[s]).wait()
        pltpu.make_async_copy(q_hbm.at[pl.ds(off, CH)], qrows[s],
                              semq[s]).wait()

        def comp(i, _):
            for j in range(8):
                sl = pl.ds(j * 16, 16)
                prows[s][i, sl] = jnp.maximum(prows[s][i, sl] + qrows[s][i, sl],
                                              0.0)
            return _
        lax.fori_loop(0, CH, comp, None)
        pltpu.make_async_copy(recv_hbm.at[pl.ds(off, CH)], ridxs[s],
                              semj[s]).wait()
        pltpu.sync_copy(prows[s], agg_sh.at[ridxs[s]], add=True)

    for c in range(NSLOT):
        issue_idx(c, c)
    issue_data(0, 0)
    issue_data(1, 1)

    nmain = (NCHUNK - NSLOT - 1) // NSLOT

    def loop_body(kk, _):
        b = kk * NSLOT
        for s in range(NSLOT):
            c = b + s
            step(c, s)
            issue_idx(c + NSLOT, s)
            issue_data(c + 2, (s + 2) % NSLOT)
        return _
    lax.fori_loop(0, nmain, loop_body, None)

    for c in range(nmain * NSLOT, NCHUNK):
        s = c % NSLOT
        step(c, s)
        if c + NSLOT < NCHUNK:
            issue_idx(c + NSLOT, s)
        if c + 2 < NCHUNK:
            issue_data(c + 2, (c + 2) % NSLOT)

    plsc.subcore_barrier()

    for r in range(ROWS_PER_SUB // ZROWS):
        row = base_row + r * ZROWS
        pltpu.sync_copy(agg_sh.at[pl.ds(row, ZROWS)], zbuf)
        pltpu.sync_copy(zbuf, out_hbm.at[cid, pl.ds(row, ZROWS)])


def _sc_aggregate(p, q, senders, receivers):
    mesh = plsc.VectorSubcoreMesh(core_axis_name="c", subcore_axis_name="s")
    scratch = (
        [pltpu.VMEM((CH,), jnp.int32) for _ in range(2 * NSLOT)]
        + [pltpu.VMEM((CH, D), jnp.float32) for _ in range(2 * NSLOT)]
        + [pltpu.VMEM((ZROWS, D), jnp.float32),
           pltpu.VMEM_SHARED((NP, D), jnp.float32)]
        + [pltpu.SemaphoreType.DMA for _ in range(4 * NSLOT)]
    )
    kern = functools.partial(
        pl.kernel,
        mesh=mesh,
        out_type=jax.ShapeDtypeStruct((NC, NP, D), jnp.float32),
        scratch_types=scratch,
    )(_sc_body)
    return kern(p, q, senders, receivers)


@jax.jit
def kernel(node_latents, edge_latents, edge_index, W_msg, b_msg, W_upd, b_upd):
    w1 = W_msg[:D]
    w2 = W_msg[D:]
    wu1 = W_upd[:D]
    wu2 = W_upd[D:]

    p = _compute_p(node_latents, w1)
    q = _compute_q(edge_latents, w2, b_msg)
    agg_partials = _sc_aggregate(p, q, edge_index[0], edge_index[1])
    new_node_latents = _compute_update(node_latents, agg_partials, wu1, wu2,
                                       b_upd)
    return (new_node_latents, edge_latents)
